# baseline (speedup 1.0000x reference)
import functools
import math

import jax
import jax.numpy as jnp
from jax.experimental import pallas as pl
from jax.experimental.pallas import tpu as pltpu

B, N, F = 2048, 20, 16
NP = 24
FC = F - 1
NE, DE = 3, 8
H = 64
OBS = 64
A = 2
TB = 16


def _gr_actor_kernel(obs_ref, nof_ref, adj_ref, adjs_ref, aid_ref, rnn_ref,
                     msk_ref, emb_ref, w1f_ref, w1e_ref, wlast_ref, b1_ref,
                     w2_ref, b2_ref, wq_ref, wk_ref, wv_ref, wet_ref, we_ref,
                     wo_ref,
                     bo_ref, aw1a_ref, aw1b_ref, ab1_ref, aw2_ref, ab2_ref,
                     mw_ref, mb_ref, ls_ref,
                     act_out, lp_out, rnn_out):
    f32 = jnp.float32

    nof = nof_ref[...].reshape(TB * NP, F)
    feat_a = jnp.dot(nof, w1f_ref[...], preferred_element_type=f32)
    ent_f = nof[:, FC:FC + 1]
    ent = jnp.clip((ent_f * NE).astype(jnp.int32), 0, NE - 1)
    ohe = (jax.lax.broadcasted_iota(jnp.int32, (TB * NP, NE), 1) == ent)
    et = jnp.dot(emb_ref[...], w1e_ref[...], preferred_element_type=f32)
    emb_a = jnp.dot(ohe.astype(f32), et, preferred_element_type=f32)
    a_node = feat_a + emb_a + b1_ref[...]

    a3 = a_node.reshape(TB, 1, NP, H)
    adjs = adjs_ref[...].reshape(TB, NP, NP, 1)
    pre1 = jnp.broadcast_to(a3, (TB, NP, NP, H)) + adjs * wlast_ref[...]
    h1 = jnp.maximum(pre1, 0.0).reshape(TB * NP * NP, H)
    m = jnp.dot(h1, w2_ref[...], preferred_element_type=f32) + b2_ref[...]
    m = jnp.maximum(m, 0.0).reshape(TB, NP, NP, H)

    wm = (adjs > 0.5).astype(f32)
    deg = jnp.sum(wm, axis=2)
    x1 = jnp.sum(m * wm, axis=2) / jnp.maximum(deg, 1.0)
    x1f = x1.reshape(TB * NP, H)

    q = jnp.dot(x1f, wq_ref[...], preferred_element_type=f32)
    k = jnp.dot(x1f, wk_ref[...], preferred_element_type=f32)
    v = jnp.dot(x1f, wv_ref[...], preferred_element_type=f32)
    qe = jnp.dot(q, wet_ref[...], preferred_element_type=f32)
    q3 = q.reshape(TB, NP, H)
    k3 = k.reshape(TB, NP, H)
    v3 = v.reshape(TB, NP, H)
    qe3 = qe.reshape(TB, NP, 1)

    adjb = adj_ref[...]
    mask = (adjb > 0.5).astype(f32)
    scores = jax.lax.dot_general(q3, k3, (((2,), (2,)), ((0,), (0,))),
                                 preferred_element_type=f32)
    scores = (scores + adjb * qe3) * (1.0 / math.sqrt(H))
    scores = jnp.where(mask > 0, scores, -1e9)
    smax = jnp.max(scores, axis=-1, keepdims=True)
    se = jnp.exp(scores - smax)
    alpha = se / jnp.sum(se, axis=-1, keepdims=True) * mask

    x2 = jax.lax.dot_general(alpha, v3, (((2,), (1,)), ((0,), (0,))),
                             preferred_element_type=f32)
    aw = jnp.sum(alpha * adjb, axis=-1, keepdims=True)
    x2 = x2 + aw * we_ref[...].reshape(1, 1, H)
    x2f = x2.reshape(TB * NP, H)
    x2o = jnp.dot(x2f, wo_ref[...], preferred_element_type=f32) + bo_ref[...]
    x2o = jnp.maximum(x2o, 0.0).reshape(TB, NP, H)

    aid = aid_ref[...]
    niota = jax.lax.broadcasted_iota(jnp.int32, (TB, NP, 1), 1)
    ohn = (niota == aid).astype(f32)
    g = jnp.sum(x2o * ohn, axis=1)

    h = (jnp.dot(obs_ref[...], aw1a_ref[...], preferred_element_type=f32)
         + jnp.dot(g, aw1b_ref[...], preferred_element_type=f32)
         + ab1_ref[...])
    h = jnp.maximum(h, 0.0)
    h = jnp.dot(h, aw2_ref[...], preferred_element_type=f32) + ab2_ref[...]
    h = jnp.maximum(h, 0.0)
    mean = jnp.dot(h, mw_ref[...], preferred_element_type=f32) + mb_ref[...]
    act_out[...] = mean

    ls = ls_ref[...]
    lp = jnp.sum(-ls) - A * 0.5 * math.log(2.0 * math.pi)
    lp_out[...] = jnp.full((TB, 1), lp, dtype=f32)

    rnn_out[...] = rnn_ref[...] * msk_ref[...].reshape(TB, 1, 1)


def kernel(obs, node_obs, adj, agent_id, rnn_states, masks, emb_table, W1, b1,
           W2, b2, Wq, Wk, Wv, We, Wo, bo, actor_W1, actor_b1, actor_W2,
           actor_b2, mean_W, mean_b, log_std):
    f32 = jnp.float32
    b = obs.shape[0]

    nof = jnp.pad(node_obs, ((0, 0), (0, NP - N), (0, 0)))
    adj_p = jnp.pad(adj, ((0, 0), (0, NP - N), (0, NP - N)))
    adj_s = adj_p.reshape(b, NP * NP, 1)
    aid = agent_id.reshape(b, 1, 1).astype(jnp.int32)
    msk = masks.reshape(b, 1, 1)
    w1f = jnp.concatenate([W1[:FC], jnp.zeros((1, H), f32)], axis=0)
    w1e = W1[FC:FC + DE]
    wlast = W1[FC + DE:FC + DE + 1]
    wet = We.T
    aw1a = actor_W1[:OBS]
    aw1b = actor_W1[OBS:]
    b1r = b1.reshape(1, H); b2r = b2.reshape(1, H); bor = bo.reshape(1, H)
    ab1 = actor_b1.reshape(1, H); ab2 = actor_b2.reshape(1, H)
    mbr = mean_b.reshape(1, A); lsr = log_std.reshape(1, A)

    grid = (b // TB,)

    def bspec(shape):
        nd = len(shape)
        return pl.BlockSpec((TB,) + shape[1:],
                            lambda i, _nd=nd: (i,) + (0,) * (_nd - 1))

    def wspec(shape):
        nd = len(shape)
        return pl.BlockSpec(shape, lambda i, _nd=nd: (0,) * _nd)

    out_shapes = (
        jax.ShapeDtypeStruct((b, A), f32),
        jax.ShapeDtypeStruct((b, 1), f32),
        jax.ShapeDtypeStruct((b, 1, H), f32),
    )
    out_specs = (bspec((b, A)), bspec((b, 1)), bspec((b, 1, H)))

    in_arrays = (obs, nof, adj_p, adj_s, aid, rnn_states, msk, emb_table,
                 w1f, w1e, wlast, b1r, W2, b2r, Wq, Wk, Wv, wet, We, Wo, bor,
                 aw1a, aw1b, ab1, actor_W2, ab2, mean_W, mbr, lsr)
    batched = {0, 1, 2, 3, 4, 5, 6}
    in_specs = [bspec(a.shape) if i in batched else wspec(a.shape)
                for i, a in enumerate(in_arrays)]

    actions, log_probs, new_rnn = pl.pallas_call(
        _gr_actor_kernel,
        grid=grid,
        in_specs=in_specs,
        out_specs=out_specs,
        out_shape=out_shapes,
        compiler_params=pltpu.CompilerParams(
            dimension_semantics=("parallel",)),
    )(*in_arrays)
    return actions, log_probs, new_rnn


# speedup vs baseline: 1.3613x; 1.3613x over previous
import functools
import math

import jax
import jax.numpy as jnp
from jax.experimental import pallas as pl
from jax.experimental.pallas import tpu as pltpu

B, N, F = 2048, 20, 16
NP = 24
FC = F - 1
NE, DE = 3, 8
H = 64
OBS = 64
A = 2
TB = 16


def _gr_actor_kernel(obs_ref, nof_ref, adj_ref, adjs_ref, aid_ref, rnn_ref,
                     msk_ref, emb_ref, w1f_ref, w1e_ref, wlast_ref, b1_ref,
                     w2_ref, b2_ref, wq_ref, wk_ref, wv_ref, wet_ref, we_ref,
                     wo_ref, bo_ref, aw1a_ref, aw1b_ref, ab1_ref, aw2_ref,
                     ab2_ref, mw_ref, mb_ref, ls_ref,
                     act_out, lp_out, rnn_out):
    f32 = jnp.float32

    nof = nof_ref[...].reshape(TB * NP, F)
    feat_a = jnp.dot(nof, w1f_ref[...], preferred_element_type=f32)
    ent_f = nof[:, FC:FC + 1]
    ent = jnp.clip((ent_f * NE).astype(jnp.int32), 0, NE - 1)
    ohe = (jax.lax.broadcasted_iota(jnp.int32, (TB * NP, NE), 1) == ent)
    et = jnp.dot(emb_ref[...], w1e_ref[...], preferred_element_type=f32)
    emb_a = jnp.dot(ohe.astype(f32), et, preferred_element_type=f32)
    a_node = feat_a + emb_a + b1_ref[...]

    a4 = a_node.reshape(TB, 1, NP, H)
    adjs4 = adjs_ref[...].reshape(TB, N, NP, 1)
    sbc = jnp.broadcast_to(adjs4, (TB, N, NP, H))
    pre1 = sbc * wlast_ref[...] + a4
    h1 = jnp.maximum(pre1, 0.0).reshape(TB * N * NP, H)
    m = jnp.dot(h1, w2_ref[...], preferred_element_type=f32) + b2_ref[...]
    m = jnp.maximum(m, 0.0).reshape(TB, N, NP, H)
    msel = jnp.where(sbc > 0.5, m, 0.0)
    msum = jnp.sum(msel, axis=2)

    adjb = adj_ref[...]
    mask = (adjb > 0.5).astype(f32)
    deg = jnp.sum(mask, axis=-1, keepdims=True)
    x1 = msum * (1.0 / jnp.maximum(deg, 1.0))

    dimn = (((2,), (0,)), ((), ()))
    q = jax.lax.dot_general(x1, wq_ref[...], dimn, preferred_element_type=f32)
    k = jax.lax.dot_general(x1, wk_ref[...], dimn, preferred_element_type=f32)
    v = jax.lax.dot_general(x1, wv_ref[...], dimn, preferred_element_type=f32)
    qe = jax.lax.dot_general(q, wet_ref[...], dimn,
                             preferred_element_type=f32)

    scores = jax.lax.dot_general(q, k, (((2,), (2,)), ((0,), (0,))),
                                 preferred_element_type=f32)
    scores = (scores + adjb * qe) * (1.0 / math.sqrt(H))
    scores = jnp.where(mask > 0, scores, -1e9)
    smax = jnp.max(scores, axis=-1, keepdims=True)
    se = jnp.exp(scores - smax)
    alpha = se / jnp.sum(se, axis=-1, keepdims=True) * mask

    x2 = jax.lax.dot_general(alpha, v, (((2,), (1,)), ((0,), (0,))),
                             preferred_element_type=f32)
    aw = jnp.sum(alpha * adjb, axis=-1, keepdims=True)
    x2 = x2 + aw * we_ref[...].reshape(1, 1, H)
    x2o = jax.lax.dot_general(x2, wo_ref[...], dimn,
                              preferred_element_type=f32) + bo_ref[...]
    x2o = jnp.maximum(x2o, 0.0)

    aid = aid_ref[...]
    niota = jax.lax.broadcasted_iota(jnp.int32, (TB, N, 1), 1)
    ohn = (niota == aid).astype(f32)
    g = jnp.sum(x2o * ohn, axis=1)

    h = (jnp.dot(obs_ref[...], aw1a_ref[...], preferred_element_type=f32)
         + jnp.dot(g, aw1b_ref[...], preferred_element_type=f32)
         + ab1_ref[...])
    h = jnp.maximum(h, 0.0)
    h = jnp.dot(h, aw2_ref[...], preferred_element_type=f32) + ab2_ref[...]
    h = jnp.maximum(h, 0.0)
    mean = jnp.dot(h, mw_ref[...], preferred_element_type=f32) + mb_ref[...]
    act_out[...] = mean

    ls = ls_ref[...]
    lp = jnp.sum(-ls) - A * 0.5 * math.log(2.0 * math.pi)
    lp_out[...] = jnp.full((TB, 1), lp, dtype=f32)

    rnn_out[...] = rnn_ref[...] * msk_ref[...].reshape(TB, 1, 1)


def kernel(obs, node_obs, adj, agent_id, rnn_states, masks, emb_table, W1, b1,
           W2, b2, Wq, Wk, Wv, We, Wo, bo, actor_W1, actor_b1, actor_W2,
           actor_b2, mean_W, mean_b, log_std):
    f32 = jnp.float32
    b = obs.shape[0]

    nof = jnp.pad(node_obs, ((0, 0), (0, NP - N), (0, 0)))
    adj_s = jnp.pad(adj, ((0, 0), (0, 0), (0, NP - N))).reshape(b, N * NP, 1)
    aid = agent_id.reshape(b, 1, 1).astype(jnp.int32)
    msk = masks.reshape(b, 1, 1)
    w1f = jnp.concatenate([W1[:FC], jnp.zeros((1, H), f32)], axis=0)
    w1e = W1[FC:FC + DE]
    wlast = W1[FC + DE:FC + DE + 1]
    wet = We.T
    aw1a = actor_W1[:OBS]
    aw1b = actor_W1[OBS:]
    b1r = b1.reshape(1, H); b2r = b2.reshape(1, H); bor = bo.reshape(1, H)
    ab1 = actor_b1.reshape(1, H); ab2 = actor_b2.reshape(1, H)
    mbr = mean_b.reshape(1, A); lsr = log_std.reshape(1, A)

    grid = (b // TB,)

    def bspec(shape):
        nd = len(shape)
        return pl.BlockSpec((TB,) + shape[1:],
                            lambda i, _nd=nd: (i,) + (0,) * (_nd - 1))

    def wspec(shape):
        nd = len(shape)
        return pl.BlockSpec(shape, lambda i, _nd=nd: (0,) * _nd)

    out_shapes = (
        jax.ShapeDtypeStruct((b, A), f32),
        jax.ShapeDtypeStruct((b, 1), f32),
        jax.ShapeDtypeStruct((b, 1, H), f32),
    )
    out_specs = (bspec((b, A)), bspec((b, 1)), bspec((b, 1, H)))

    in_arrays = (obs, nof, adj, adj_s, aid, rnn_states, msk, emb_table,
                 w1f, w1e, wlast, b1r, W2, b2r, Wq, Wk, Wv, wet, We, Wo, bor,
                 aw1a, aw1b, ab1, actor_W2, ab2, mean_W, mbr, lsr)
    batched = {0, 1, 2, 3, 4, 5, 6}
    in_specs = [bspec(a.shape) if i in batched else wspec(a.shape)
                for i, a in enumerate(in_arrays)]

    actions, log_probs, new_rnn = pl.pallas_call(
        _gr_actor_kernel,
        grid=grid,
        in_specs=in_specs,
        out_specs=out_specs,
        out_shape=out_shapes,
        compiler_params=pltpu.CompilerParams(
            dimension_semantics=("parallel",)),
    )(*in_arrays)
    return actions, log_probs, new_rnn


# speedup vs baseline: 1.5149x; 1.1129x over previous
import functools
import math

import jax
import jax.numpy as jnp
from jax.experimental import pallas as pl
from jax.experimental.pallas import tpu as pltpu

B, N, F = 2048, 20, 16
NP = 24
FC = F - 1
NE, DE = 3, 8
H = 64
OBS = 64
A = 2
TB = 32


def _gr_actor_kernel(obs_ref, nof_ref, adj_ref, adjs_ref, aid_ref, rnn_ref,
                     msk_ref, emb_ref, w1f_ref, w1e_ref, wlast_ref, b1_ref,
                     w2_ref, b2_ref, wq_ref, wk_ref, wv_ref, wet_ref, we_ref,
                     wo_ref, bo_ref, aw1a_ref, aw1b_ref, ab1_ref, aw2_ref,
                     ab2_ref, mw_ref, mb_ref, ls_ref,
                     act_out, lp_out, rnn_out):
    f32 = jnp.float32

    nof = nof_ref[...].reshape(TB * NP, F)
    feat_a = jnp.dot(nof, w1f_ref[...], preferred_element_type=f32)
    ent_f = nof[:, FC:FC + 1]
    ent = jnp.clip((ent_f * NE).astype(jnp.int32), 0, NE - 1)
    ohe = (jax.lax.broadcasted_iota(jnp.int32, (TB * NP, NE), 1) == ent)
    et = jnp.dot(emb_ref[...], w1e_ref[...], preferred_element_type=f32)
    emb_a = jnp.dot(ohe.astype(f32), et, preferred_element_type=f32)
    a_node = feat_a + emb_a + b1_ref[...]

    a4 = a_node.reshape(TB, 1, NP, H)
    adjs4 = adjs_ref[...].reshape(TB, N, NP, 1)
    sbc = jnp.broadcast_to(adjs4, (TB, N, NP, H))
    pre1 = sbc * wlast_ref[...] + a4
    h1 = jnp.maximum(pre1, 0.0).reshape(TB * N * NP, H)
    m = jnp.dot(h1, w2_ref[...], preferred_element_type=f32) + b2_ref[...]
    m = jnp.maximum(m, 0.0).reshape(TB, N, NP, H)
    msel = jnp.where(sbc > 0.5, m, 0.0)
    msum = jnp.sum(msel, axis=2)

    adjb = adj_ref[...]
    mask = (adjb > 0.5).astype(f32)
    deg = jnp.sum(mask, axis=-1, keepdims=True)
    x1 = msum * (1.0 / jnp.maximum(deg, 1.0))

    dimn = (((2,), (0,)), ((), ()))
    q = jax.lax.dot_general(x1, wq_ref[...], dimn, preferred_element_type=f32)
    k = jax.lax.dot_general(x1, wk_ref[...], dimn, preferred_element_type=f32)
    v = jax.lax.dot_general(x1, wv_ref[...], dimn, preferred_element_type=f32)
    qe = jax.lax.dot_general(q, wet_ref[...], dimn,
                             preferred_element_type=f32)

    scores = jax.lax.dot_general(q, k, (((2,), (2,)), ((0,), (0,))),
                                 preferred_element_type=f32)
    scores = (scores + adjb * qe) * (1.0 / math.sqrt(H))
    scores = jnp.where(mask > 0, scores, -1e9)
    smax = jnp.max(scores, axis=-1, keepdims=True)
    se = jnp.exp(scores - smax)
    alpha = se / jnp.sum(se, axis=-1, keepdims=True) * mask

    x2 = jax.lax.dot_general(alpha, v, (((2,), (1,)), ((0,), (0,))),
                             preferred_element_type=f32)
    aw = jnp.sum(alpha * adjb, axis=-1, keepdims=True)
    x2 = x2 + aw * we_ref[...].reshape(1, 1, H)
    x2o = jax.lax.dot_general(x2, wo_ref[...], dimn,
                              preferred_element_type=f32) + bo_ref[...]
    x2o = jnp.maximum(x2o, 0.0)

    aid = aid_ref[...]
    niota = jax.lax.broadcasted_iota(jnp.int32, (TB, N, 1), 1)
    ohn = (niota == aid).astype(f32)
    g = jnp.sum(x2o * ohn, axis=1)

    h = (jnp.dot(obs_ref[...], aw1a_ref[...], preferred_element_type=f32)
         + jnp.dot(g, aw1b_ref[...], preferred_element_type=f32)
         + ab1_ref[...])
    h = jnp.maximum(h, 0.0)
    h = jnp.dot(h, aw2_ref[...], preferred_element_type=f32) + ab2_ref[...]
    h = jnp.maximum(h, 0.0)
    mean = jnp.dot(h, mw_ref[...], preferred_element_type=f32) + mb_ref[...]
    act_out[...] = mean

    ls = ls_ref[...]
    lp = jnp.sum(-ls) - A * 0.5 * math.log(2.0 * math.pi)
    lp_out[...] = jnp.full((TB, 1), lp, dtype=f32)

    rnn_out[...] = rnn_ref[...] * msk_ref[...].reshape(TB, 1, 1)


def kernel(obs, node_obs, adj, agent_id, rnn_states, masks, emb_table, W1, b1,
           W2, b2, Wq, Wk, Wv, We, Wo, bo, actor_W1, actor_b1, actor_W2,
           actor_b2, mean_W, mean_b, log_std):
    f32 = jnp.float32
    b = obs.shape[0]

    nof = jnp.pad(node_obs, ((0, 0), (0, NP - N), (0, 0)))
    adj_s = jnp.pad(adj, ((0, 0), (0, 0), (0, NP - N))).reshape(b, N * NP, 1)
    aid = agent_id.reshape(b, 1, 1).astype(jnp.int32)
    msk = masks.reshape(b, 1, 1)
    w1f = jnp.concatenate([W1[:FC], jnp.zeros((1, H), f32)], axis=0)
    w1e = W1[FC:FC + DE]
    wlast = W1[FC + DE:FC + DE + 1]
    wet = We.T
    aw1a = actor_W1[:OBS]
    aw1b = actor_W1[OBS:]
    b1r = b1.reshape(1, H); b2r = b2.reshape(1, H); bor = bo.reshape(1, H)
    ab1 = actor_b1.reshape(1, H); ab2 = actor_b2.reshape(1, H)
    mbr = mean_b.reshape(1, A); lsr = log_std.reshape(1, A)

    grid = (b // TB,)

    def bspec(shape):
        nd = len(shape)
        return pl.BlockSpec((TB,) + shape[1:],
                            lambda i, _nd=nd: (i,) + (0,) * (_nd - 1))

    def wspec(shape):
        nd = len(shape)
        return pl.BlockSpec(shape, lambda i, _nd=nd: (0,) * _nd)

    out_shapes = (
        jax.ShapeDtypeStruct((b, A), f32),
        jax.ShapeDtypeStruct((b, 1), f32),
        jax.ShapeDtypeStruct((b, 1, H), f32),
    )
    out_specs = (bspec((b, A)), bspec((b, 1)), bspec((b, 1, H)))

    in_arrays = (obs, nof, adj, adj_s, aid, rnn_states, msk, emb_table,
                 w1f, w1e, wlast, b1r, W2, b2r, Wq, Wk, Wv, wet, We, Wo, bor,
                 aw1a, aw1b, ab1, actor_W2, ab2, mean_W, mbr, lsr)
    batched = {0, 1, 2, 3, 4, 5, 6}
    in_specs = [bspec(a.shape) if i in batched else wspec(a.shape)
                for i, a in enumerate(in_arrays)]

    actions, log_probs, new_rnn = pl.pallas_call(
        _gr_actor_kernel,
        grid=grid,
        in_specs=in_specs,
        out_specs=out_specs,
        out_shape=out_shapes,
        compiler_params=pltpu.CompilerParams(
            dimension_semantics=("parallel",)),
    )(*in_arrays)
    return actions, log_probs, new_rnn


# speedup vs baseline: 1.6076x; 1.0612x over previous
import functools
import math

import jax
import jax.numpy as jnp
from jax.experimental import pallas as pl
from jax.experimental.pallas import tpu as pltpu

B, N, F = 2048, 20, 16
NP = 24
FC = F - 1
NE, DE = 3, 8
H = 64
OBS = 64
A = 2
TB = 64


def _gr_actor_kernel(obs_ref, nof_ref, adj_ref, adjs_ref, aid_ref, rnn_ref,
                     msk_ref, emb_ref, w1f_ref, w1e_ref, wlast_ref, b1_ref,
                     w2_ref, b2_ref, wq_ref, wk_ref, wv_ref, wet_ref, we_ref,
                     wo_ref, bo_ref, aw1a_ref, aw1b_ref, ab1_ref, aw2_ref,
                     ab2_ref, mw_ref, mb_ref, ls_ref,
                     act_out, lp_out, rnn_out):
    f32 = jnp.float32

    nof = nof_ref[...].reshape(TB * NP, F)
    feat_a = jnp.dot(nof, w1f_ref[...], preferred_element_type=f32)
    ent_f = nof[:, FC:FC + 1]
    ent = jnp.clip((ent_f * NE).astype(jnp.int32), 0, NE - 1)
    ohe = (jax.lax.broadcasted_iota(jnp.int32, (TB * NP, NE), 1) == ent)
    et = jnp.dot(emb_ref[...], w1e_ref[...], preferred_element_type=f32)
    emb_a = jnp.dot(ohe.astype(f32), et, preferred_element_type=f32)
    a_node = feat_a + emb_a + b1_ref[...]

    a4 = a_node.reshape(TB, 1, NP, H)
    adjs4 = adjs_ref[...].reshape(TB, N, NP, 1)
    sbc = jnp.broadcast_to(adjs4, (TB, N, NP, H))
    pre1 = sbc * wlast_ref[...] + a4
    h1 = jnp.maximum(pre1, 0.0).reshape(TB * N * NP, H)
    m = jnp.dot(h1, w2_ref[...], preferred_element_type=f32) + b2_ref[...]
    m = jnp.maximum(m, 0.0).reshape(TB, N, NP, H)
    msel = jnp.where(sbc > 0.5, m, 0.0)
    msum = jnp.sum(msel, axis=2)

    adjb = adj_ref[...]
    mask = (adjb > 0.5).astype(f32)
    deg = jnp.sum(mask, axis=-1, keepdims=True)
    x1 = msum * (1.0 / jnp.maximum(deg, 1.0))

    dimn = (((2,), (0,)), ((), ()))
    q = jax.lax.dot_general(x1, wq_ref[...], dimn, preferred_element_type=f32)
    k = jax.lax.dot_general(x1, wk_ref[...], dimn, preferred_element_type=f32)
    v = jax.lax.dot_general(x1, wv_ref[...], dimn, preferred_element_type=f32)
    qe = jax.lax.dot_general(q, wet_ref[...], dimn,
                             preferred_element_type=f32)

    scores = jax.lax.dot_general(q, k, (((2,), (2,)), ((0,), (0,))),
                                 preferred_element_type=f32)
    scores = (scores + adjb * qe) * (1.0 / math.sqrt(H))
    scores = jnp.where(mask > 0, scores, -1e9)
    smax = jnp.max(scores, axis=-1, keepdims=True)
    se = jnp.exp(scores - smax)
    alpha = se / jnp.sum(se, axis=-1, keepdims=True) * mask

    x2 = jax.lax.dot_general(alpha, v, (((2,), (1,)), ((0,), (0,))),
                             preferred_element_type=f32)
    aw = jnp.sum(alpha * adjb, axis=-1, keepdims=True)
    x2 = x2 + aw * we_ref[...].reshape(1, 1, H)
    x2o = jax.lax.dot_general(x2, wo_ref[...], dimn,
                              preferred_element_type=f32) + bo_ref[...]
    x2o = jnp.maximum(x2o, 0.0)

    aid = aid_ref[...]
    niota = jax.lax.broadcasted_iota(jnp.int32, (TB, N, 1), 1)
    ohn = (niota == aid).astype(f32)
    g = jnp.sum(x2o * ohn, axis=1)

    h = (jnp.dot(obs_ref[...], aw1a_ref[...], preferred_element_type=f32)
         + jnp.dot(g, aw1b_ref[...], preferred_element_type=f32)
         + ab1_ref[...])
    h = jnp.maximum(h, 0.0)
    h = jnp.dot(h, aw2_ref[...], preferred_element_type=f32) + ab2_ref[...]
    h = jnp.maximum(h, 0.0)
    mean = jnp.dot(h, mw_ref[...], preferred_element_type=f32) + mb_ref[...]
    act_out[...] = mean

    ls = ls_ref[...]
    lp = jnp.sum(-ls) - A * 0.5 * math.log(2.0 * math.pi)
    lp_out[...] = jnp.full((TB, 1), lp, dtype=f32)

    rnn_out[...] = rnn_ref[...] * msk_ref[...].reshape(TB, 1, 1)


def kernel(obs, node_obs, adj, agent_id, rnn_states, masks, emb_table, W1, b1,
           W2, b2, Wq, Wk, Wv, We, Wo, bo, actor_W1, actor_b1, actor_W2,
           actor_b2, mean_W, mean_b, log_std):
    f32 = jnp.float32
    b = obs.shape[0]

    nof = jnp.pad(node_obs, ((0, 0), (0, NP - N), (0, 0)))
    adj_s = jnp.pad(adj, ((0, 0), (0, 0), (0, NP - N))).reshape(b, N * NP, 1)
    aid = agent_id.reshape(b, 1, 1).astype(jnp.int32)
    msk = masks.reshape(b, 1, 1)
    w1f = jnp.concatenate([W1[:FC], jnp.zeros((1, H), f32)], axis=0)
    w1e = W1[FC:FC + DE]
    wlast = W1[FC + DE:FC + DE + 1]
    wet = We.T
    aw1a = actor_W1[:OBS]
    aw1b = actor_W1[OBS:]
    b1r = b1.reshape(1, H); b2r = b2.reshape(1, H); bor = bo.reshape(1, H)
    ab1 = actor_b1.reshape(1, H); ab2 = actor_b2.reshape(1, H)
    mbr = mean_b.reshape(1, A); lsr = log_std.reshape(1, A)

    grid = (b // TB,)

    def bspec(shape):
        nd = len(shape)
        return pl.BlockSpec((TB,) + shape[1:],
                            lambda i, _nd=nd: (i,) + (0,) * (_nd - 1))

    def wspec(shape):
        nd = len(shape)
        return pl.BlockSpec(shape, lambda i, _nd=nd: (0,) * _nd)

    out_shapes = (
        jax.ShapeDtypeStruct((b, A), f32),
        jax.ShapeDtypeStruct((b, 1), f32),
        jax.ShapeDtypeStruct((b, 1, H), f32),
    )
    out_specs = (bspec((b, A)), bspec((b, 1)), bspec((b, 1, H)))

    in_arrays = (obs, nof, adj, adj_s, aid, rnn_states, msk, emb_table,
                 w1f, w1e, wlast, b1r, W2, b2r, Wq, Wk, Wv, wet, We, Wo, bor,
                 aw1a, aw1b, ab1, actor_W2, ab2, mean_W, mbr, lsr)
    batched = {0, 1, 2, 3, 4, 5, 6}
    in_specs = [bspec(a.shape) if i in batched else wspec(a.shape)
                for i, a in enumerate(in_arrays)]

    actions, log_probs, new_rnn = pl.pallas_call(
        _gr_actor_kernel,
        grid=grid,
        in_specs=in_specs,
        out_specs=out_specs,
        out_shape=out_shapes,
        compiler_params=pltpu.CompilerParams(
            dimension_semantics=("parallel",)),
    )(*in_arrays)
    return actions, log_probs, new_rnn
